# baseline (speedup 1.0000x reference)
import functools
import math

import jax
import jax.numpy as jnp
from jax import lax
from jax.experimental import pallas as pl
from jax.experimental.pallas import tpu as pltpu
from jax.experimental.pallas import tpu_sc as plsc

N = 16384
RATIO = 0.1
K = 5
M = math.ceil(RATIO * N)
ROWS = 128
COLS = N // ROWS
CBLK = 128
NBLK = (M + CBLK - 1) // CBLK
MPAD = NBLK * CBLK
GTOT = M * K
GCHUNK = 128
NCH = (GTOT + GCHUNK - 1) // GCHUNK
GPAD = NCH * GCHUNK
NWORK = 32


def _fps_body(px_ref, py_ref, pz_ref, pxs_ref, pys_ref, pzs_ref,
              idx_out, cx_out, cy_out, cz_out, lin_ref):
    px = px_ref[...]
    py = py_ref[...]
    pz = pz_ref[...]
    cx0 = px[0, 0]
    cy0 = py[0, 0]
    cz0 = pz[0, 0]
    idx_out[0] = jnp.int32(0)
    cx_out[0] = cx0
    cy_out[0] = cy0
    cz_out[0] = cz0
    dx = px - cx0
    dy = py - cy0
    dz = pz - cz0
    d0 = dx * dx + dy * dy + dz * dz

    def padbody(i, carry):
        idx_out[i] = jnp.int32(0)
        cx_out[i] = jnp.float32(0.0)
        cy_out[i] = jnp.float32(0.0)
        cz_out[i] = jnp.float32(0.0)
        return carry

    lax.fori_loop(M, MPAD, padbody, jnp.int32(0))
    lin_ref[...] = (lax.broadcasted_iota(jnp.int32, (ROWS, COLS), 0) * COLS
                    + lax.broadcasted_iota(jnp.int32, (ROWS, COLS), 1)
                    ).astype(jnp.float32)

    def body(i, d):
        va = d
        ia = lin_ref[...]
        size = ROWS
        while size > 1:
            h = size // 2
            va1, va2 = va[:h], va[h:]
            ia1, ia2 = ia[:h], ia[h:]
            win = (va1 > va2) | ((va1 == va2) & (ia1 < ia2))
            va = jnp.where(win, va1, va2)
            ia = jnp.where(win, ia1, ia2)
            size = h
        mxk = jnp.max(va, axis=1, keepdims=True)
        cand = jnp.where(va == mxk, ia, jnp.float32(3.0e7))
        nxt = jnp.min(cand).astype(jnp.int32)
        idx_out[i] = nxt
        cx = pxs_ref[nxt]
        cy = pys_ref[nxt]
        cz = pzs_ref[nxt]
        cx_out[i] = cx
        cy_out[i] = cy
        cz_out[i] = cz
        ddx = px_ref[...] - cx
        ddy = py_ref[...] - cy
        ddz = pz_ref[...] - cz
        dd = ddx * ddx + ddy * ddy + ddz * ddz
        return jnp.minimum(d, dd)

    lax.fori_loop(1, M, body, d0, unroll=2)


def _fps_call(pxm, pym, pzm, pxs, pys, pzs):
    out_shape = [
        jax.ShapeDtypeStruct((MPAD,), jnp.int32),
        jax.ShapeDtypeStruct((MPAD,), jnp.float32),
        jax.ShapeDtypeStruct((MPAD,), jnp.float32),
        jax.ShapeDtypeStruct((MPAD,), jnp.float32),
    ]
    return pl.pallas_call(
        _fps_body,
        out_shape=out_shape,
        in_specs=[
            pl.BlockSpec((ROWS, COLS), lambda: (0, 0)),
            pl.BlockSpec((ROWS, COLS), lambda: (0, 0)),
            pl.BlockSpec((ROWS, COLS), lambda: (0, 0)),
            pl.BlockSpec(memory_space=pltpu.SMEM),
            pl.BlockSpec(memory_space=pltpu.SMEM),
            pl.BlockSpec(memory_space=pltpu.SMEM),
        ],
        out_specs=[pl.BlockSpec(memory_space=pltpu.SMEM)] * 4,
        scratch_shapes=[pltpu.VMEM((ROWS, COLS), jnp.float32)],
    )(pxm, pym, pzm, pxs, pys, pzs)


def _knn_body(cx_ref, cy_ref, cz_ref, px_ref, py_ref, pz_ref, out_ref, d2_ref):
    cx = jnp.reshape(cx_ref[...], (CBLK, 1))
    cy = jnp.reshape(cy_ref[...], (CBLK, 1))
    cz = jnp.reshape(cz_ref[...], (CBLK, 1))
    px = px_ref[...]
    py = py_ref[...]
    pz = pz_ref[...]
    dx = cx - px
    dy = cy - py
    dz = cz - pz
    d2 = dx * dx + dy * dy + dz * dz
    d2_ref[...] = d2
    mv = jnp.min(d2, axis=1, keepdims=True)

    iotaf = lax.broadcasted_iota(jnp.int32, (CBLK, N), 1).astype(jnp.float32)
    li = lax.broadcasted_iota(jnp.int32, (CBLK, 8), 1)
    acc = jnp.zeros((CBLK, 8), jnp.int32)
    for k in range(K):
        d2 = d2_ref[...]
        cand = jnp.where(d2 == mv, iotaf, jnp.float32(3.0e7))
        ikf = jnp.min(cand, axis=1, keepdims=True)
        acc = jnp.where(li == k, ikf.astype(jnp.int32), acc)
        if k < K - 1:
            masked = jnp.where(iotaf == ikf, jnp.float32(jnp.inf), d2)
            d2_ref[...] = masked
            mv = jnp.min(masked, axis=1, keepdims=True)
    out_ref[0] = acc


def _knn_call(cxp, cyp, czp, px1, py1, pz1):
    grid = (NBLK,)
    cen_spec = pl.BlockSpec((1, 1, CBLK), lambda b: (b, 0, 0))
    pts_spec = pl.BlockSpec((1, N), lambda b: (0, 0))
    return pl.pallas_call(
        _knn_body,
        grid=grid,
        in_specs=[cen_spec, cen_spec, cen_spec, pts_spec, pts_spec, pts_spec],
        out_specs=pl.BlockSpec((1, CBLK, 8), lambda b: (b, 0, 0)),
        out_shape=jax.ShapeDtypeStruct((NBLK, CBLK, 8), jnp.int32),
        scratch_shapes=[pltpu.VMEM((CBLK, N), jnp.float32)],
    )(cxp, cyp, czp, px1, py1, pz1)


CCH = MPAD // GCHUNK
TCH = CCH + NCH


def _gather_body(cidx_hbm, gidx_hbm, tx_hbm, ty_hbm, tz_hbm,
                 cx_hbm, cy_hbm, cz_hbm, gx_hbm, gy_hbm, gz_hbm,
                 idx_v, row_v, sem):
    wid = lax.axis_index("s") * 2 + lax.axis_index("c")

    def do_chunk(idx_hbm, outs, c):
        base = c * GCHUNK
        pltpu.sync_copy(idx_hbm.at[pl.ds(base, GCHUNK)], idx_v)
        for t_hbm, o_hbm in zip((tx_hbm, ty_hbm, tz_hbm), outs):
            pltpu.async_copy(t_hbm.at[idx_v], row_v, sem).wait()
            pltpu.sync_copy(row_v, o_hbm.at[pl.ds(base, GCHUNK)])

    for r in range((TCH + NWORK - 1) // NWORK):
        c = wid + r * NWORK

        @pl.when(c < CCH)
        def _():
            do_chunk(cidx_hbm, (cx_hbm, cy_hbm, cz_hbm), c)

        @pl.when((c >= CCH) & (c < TCH))
        def _():
            do_chunk(gidx_hbm, (gx_hbm, gy_hbm, gz_hbm), c - CCH)


def _gather_call(cidx, gidx, px, py, pz):
    mesh = plsc.VectorSubcoreMesh(core_axis_name="c", subcore_axis_name="s")
    f = pl.kernel(
        _gather_body,
        out_type=[jax.ShapeDtypeStruct((MPAD,), jnp.float32)] * 3
        + [jax.ShapeDtypeStruct((GPAD,), jnp.float32)] * 3,
        mesh=mesh,
        scratch_types=[
            pltpu.VMEM((GCHUNK,), jnp.int32),
            pltpu.VMEM((GCHUNK,), jnp.float32),
            pltpu.SemaphoreType.DMA,
        ],
    )
    return f(cidx, gidx, px, py, pz)


def kernel(x, pos, batch):
    px = pos[:, 0]
    py = pos[:, 1]
    pz = pos[:, 2]
    pxm = px.reshape(ROWS, COLS)
    pym = py.reshape(ROWS, COLS)
    pzm = pz.reshape(ROWS, COLS)

    fps_idx, cx, cy, cz = _fps_call(pxm, pym, pzm, px, py, pz)

    cxp = cx.reshape(NBLK, 1, CBLK)
    cyp = cy.reshape(NBLK, 1, CBLK)
    czp = cz.reshape(NBLK, 1, CBLK)

    nbr8 = _knn_call(cxp, cyp, czp,
                     px.reshape(1, N), py.reshape(1, N), pz.reshape(1, N))
    gidx = nbr8[:, :, :K].reshape(GPAD)

    ccx, ccy, ccz, ggx, ggy, ggz = _gather_call(fps_idx, gidx, px, py, pz)
    centroids = jnp.stack([ccx[:M], ccy[:M], ccz[:M]], axis=1)
    groups = jnp.stack([ggx[:GTOT], ggy[:GTOT], ggz[:GTOT]], axis=1)
    return centroids, groups


# speedup vs baseline: 1.0061x; 1.0061x over previous
import functools
import math

import jax
import jax.numpy as jnp
from jax import lax
from jax.experimental import pallas as pl
from jax.experimental.pallas import tpu as pltpu
from jax.experimental.pallas import tpu_sc as plsc

N = 16384
RATIO = 0.1
K = 5
M = math.ceil(RATIO * N)
ROWS = 128
COLS = N // ROWS
CBLK = 128
NBLK = (M + CBLK - 1) // CBLK
MPAD = NBLK * CBLK
GTOT = M * K
GCHUNK = 128
NCH = (GTOT + GCHUNK - 1) // GCHUNK
GPAD = NCH * GCHUNK
NWORK = 32


def _fps_body(px_ref, py_ref, pz_ref, pxs_ref, pys_ref, pzs_ref,
              idx_out, cx_out, cy_out, cz_out, lin_ref):
    px = px_ref[...]
    py = py_ref[...]
    pz = pz_ref[...]
    cx0 = px[0, 0]
    cy0 = py[0, 0]
    cz0 = pz[0, 0]
    idx_out[0] = jnp.int32(0)
    cx_out[0] = cx0
    cy_out[0] = cy0
    cz_out[0] = cz0
    dx = px - cx0
    dy = py - cy0
    dz = pz - cz0
    d0 = dx * dx + dy * dy + dz * dz

    def padbody(i, carry):
        idx_out[i] = jnp.int32(0)
        cx_out[i] = jnp.float32(0.0)
        cy_out[i] = jnp.float32(0.0)
        cz_out[i] = jnp.float32(0.0)
        return carry

    lax.fori_loop(M, MPAD, padbody, jnp.int32(0))
    lin_ref[...] = (lax.broadcasted_iota(jnp.int32, (ROWS, COLS), 0) * COLS
                    + lax.broadcasted_iota(jnp.int32, (ROWS, COLS), 1)
                    ).astype(jnp.float32)

    def body(i, d):
        va = d
        ia = lin_ref[...]
        size = ROWS
        while size > 1:
            h = size // 2
            va1, va2 = va[:h], va[h:]
            ia1, ia2 = ia[:h], ia[h:]
            win = (va1 > va2) | ((va1 == va2) & (ia1 < ia2))
            va = jnp.where(win, va1, va2)
            ia = jnp.where(win, ia1, ia2)
            size = h
        mxk = jnp.max(va, axis=1, keepdims=True)
        cand = jnp.where(va == mxk, ia, jnp.float32(3.0e7))
        nxt = jnp.min(cand).astype(jnp.int32)
        idx_out[i] = nxt
        cx = pxs_ref[nxt]
        cy = pys_ref[nxt]
        cz = pzs_ref[nxt]
        cx_out[i] = cx
        cy_out[i] = cy
        cz_out[i] = cz
        ddx = px_ref[...] - cx
        ddy = py_ref[...] - cy
        ddz = pz_ref[...] - cz
        dd = ddx * ddx + ddy * ddy + ddz * ddz
        return jnp.minimum(d, dd)

    lax.fori_loop(1, M, body, d0)


def _fps_call(pxm, pym, pzm, pxs, pys, pzs):
    out_shape = [
        jax.ShapeDtypeStruct((MPAD,), jnp.int32),
        jax.ShapeDtypeStruct((MPAD,), jnp.float32),
        jax.ShapeDtypeStruct((MPAD,), jnp.float32),
        jax.ShapeDtypeStruct((MPAD,), jnp.float32),
    ]
    return pl.pallas_call(
        _fps_body,
        out_shape=out_shape,
        in_specs=[
            pl.BlockSpec((ROWS, COLS), lambda: (0, 0)),
            pl.BlockSpec((ROWS, COLS), lambda: (0, 0)),
            pl.BlockSpec((ROWS, COLS), lambda: (0, 0)),
            pl.BlockSpec(memory_space=pltpu.SMEM),
            pl.BlockSpec(memory_space=pltpu.SMEM),
            pl.BlockSpec(memory_space=pltpu.SMEM),
        ],
        out_specs=[pl.BlockSpec(memory_space=pltpu.SMEM)] * 4,
        scratch_shapes=[pltpu.VMEM((ROWS, COLS), jnp.float32)],
    )(pxm, pym, pzm, pxs, pys, pzs)


def _knn_body(cx_ref, cy_ref, cz_ref, px_ref, py_ref, pz_ref, out_ref, d2_ref):
    cx = jnp.reshape(cx_ref[...], (CBLK, 1))
    cy = jnp.reshape(cy_ref[...], (CBLK, 1))
    cz = jnp.reshape(cz_ref[...], (CBLK, 1))
    px = px_ref[...]
    py = py_ref[...]
    pz = pz_ref[...]
    dx = cx - px
    dy = cy - py
    dz = cz - pz
    d2v = dx * dx + dy * dy + dz * dz
    mv = jnp.min(d2v, axis=1, keepdims=True)

    iotaf = lax.broadcasted_iota(jnp.int32, (CBLK, N), 1).astype(jnp.float32)
    li = lax.broadcasted_iota(jnp.int32, (CBLK, 8), 1)
    acc = jnp.zeros((CBLK, 8), jnp.int32)
    for k in range(K):
        d2 = d2v if k == 0 else d2_ref[...]
        cand = jnp.where(d2 == mv, iotaf, jnp.float32(3.0e7))
        ikf = jnp.min(cand, axis=1, keepdims=True)
        acc = jnp.where(li == k, ikf.astype(jnp.int32), acc)
        if k < K - 1:
            masked = jnp.where(iotaf == ikf, jnp.float32(jnp.inf), d2)
            d2_ref[...] = masked
            mv = jnp.min(masked, axis=1, keepdims=True)
    out_ref[0] = acc


def _knn_call(cxp, cyp, czp, px1, py1, pz1):
    grid = (NBLK,)
    cen_spec = pl.BlockSpec((1, 1, CBLK), lambda b: (b, 0, 0))
    pts_spec = pl.BlockSpec((1, N), lambda b: (0, 0))
    return pl.pallas_call(
        _knn_body,
        grid=grid,
        in_specs=[cen_spec, cen_spec, cen_spec, pts_spec, pts_spec, pts_spec],
        out_specs=pl.BlockSpec((1, CBLK, 8), lambda b: (b, 0, 0)),
        out_shape=jax.ShapeDtypeStruct((NBLK, CBLK, 8), jnp.int32),
        scratch_shapes=[pltpu.VMEM((CBLK, N), jnp.float32)],
    )(cxp, cyp, czp, px1, py1, pz1)


CCH = MPAD // GCHUNK
TCH = CCH + NCH


def _gather_body(cidx_hbm, gidx_hbm, tx_hbm, ty_hbm, tz_hbm,
                 cx_hbm, cy_hbm, cz_hbm, gx_hbm, gy_hbm, gz_hbm,
                 idx_v, row_v, sem):
    wid = lax.axis_index("s") * 2 + lax.axis_index("c")

    def do_chunk(idx_hbm, outs, c):
        base = c * GCHUNK
        pltpu.sync_copy(idx_hbm.at[pl.ds(base, GCHUNK)], idx_v)
        for t_hbm, o_hbm in zip((tx_hbm, ty_hbm, tz_hbm), outs):
            pltpu.async_copy(t_hbm.at[idx_v], row_v, sem).wait()
            pltpu.sync_copy(row_v, o_hbm.at[pl.ds(base, GCHUNK)])

    for r in range((TCH + NWORK - 1) // NWORK):
        c = wid + r * NWORK

        @pl.when(c < CCH)
        def _():
            do_chunk(cidx_hbm, (cx_hbm, cy_hbm, cz_hbm), c)

        @pl.when((c >= CCH) & (c < TCH))
        def _():
            do_chunk(gidx_hbm, (gx_hbm, gy_hbm, gz_hbm), c - CCH)


def _gather_call(cidx, gidx, px, py, pz):
    mesh = plsc.VectorSubcoreMesh(core_axis_name="c", subcore_axis_name="s")
    f = pl.kernel(
        _gather_body,
        out_type=[jax.ShapeDtypeStruct((MPAD,), jnp.float32)] * 3
        + [jax.ShapeDtypeStruct((GPAD,), jnp.float32)] * 3,
        mesh=mesh,
        scratch_types=[
            pltpu.VMEM((GCHUNK,), jnp.int32),
            pltpu.VMEM((GCHUNK,), jnp.float32),
            pltpu.SemaphoreType.DMA,
        ],
    )
    return f(cidx, gidx, px, py, pz)


def kernel(x, pos, batch):
    px = pos[:, 0]
    py = pos[:, 1]
    pz = pos[:, 2]
    pxm = px.reshape(ROWS, COLS)
    pym = py.reshape(ROWS, COLS)
    pzm = pz.reshape(ROWS, COLS)

    fps_idx, cx, cy, cz = _fps_call(pxm, pym, pzm, px, py, pz)

    cxp = cx.reshape(NBLK, 1, CBLK)
    cyp = cy.reshape(NBLK, 1, CBLK)
    czp = cz.reshape(NBLK, 1, CBLK)

    nbr8 = _knn_call(cxp, cyp, czp,
                     px.reshape(1, N), py.reshape(1, N), pz.reshape(1, N))
    gidx = nbr8[:, :, :K].reshape(GPAD)

    ccx, ccy, ccz, ggx, ggy, ggz = _gather_call(fps_idx, gidx, px, py, pz)
    centroids = jnp.stack([ccx[:M], ccy[:M], ccz[:M]], axis=1)
    groups = jnp.stack([ggx[:GTOT], ggy[:GTOT], ggz[:GTOT]], axis=1)
    return centroids, groups


# speedup vs baseline: 1.0137x; 1.0075x over previous
import functools
import math

import jax
import jax.numpy as jnp
from jax import lax
from jax.experimental import pallas as pl
from jax.experimental.pallas import tpu as pltpu
from jax.experimental.pallas import tpu_sc as plsc

N = 16384
RATIO = 0.1
K = 5
M = math.ceil(RATIO * N)
ROWS = 128
COLS = N // ROWS
CBLK = 128
NBLK = (M + CBLK - 1) // CBLK
MPAD = NBLK * CBLK
GTOT = M * K
GCHUNK = 128
NCH = (GTOT + GCHUNK - 1) // GCHUNK
GPAD = NCH * GCHUNK
NWORK = 32


def _fpsknn_body(px_ref, py_ref, pz_ref, pxs_ref, pys_ref, pzs_ref,
                 px1_ref, py1_ref, pz1_ref,
                 idx_out, nbr_out, d2_ref, cen_ref, lin_ref):
    b = pl.program_id(0)

    @pl.when(b == 0)
    def _fps():
        px = px_ref[...]
        py = py_ref[...]
        pz = pz_ref[...]
        cen_ref[...] = jnp.zeros((3, 16, COLS), jnp.float32)
        lane = lax.broadcasted_iota(jnp.int32, (1, COLS), 1)
        cx0 = px[0, 0]
        cy0 = py[0, 0]
        cz0 = pz[0, 0]
        idx_out[0] = jnp.int32(0)
        lm0 = lane == 0
        cen_ref[0, pl.ds(0, 1), :] = jnp.where(lm0, cx0, 0.0)
        cen_ref[1, pl.ds(0, 1), :] = jnp.where(lm0, cy0, 0.0)
        cen_ref[2, pl.ds(0, 1), :] = jnp.where(lm0, cz0, 0.0)
        dx = px - cx0
        dy = py - cy0
        dz = pz - cz0
        d0 = dx * dx + dy * dy + dz * dz

        def padbody(i, carry):
            idx_out[i] = jnp.int32(0)
            return carry

        lax.fori_loop(M, MPAD, padbody, jnp.int32(0))
        lin_ref[...] = (lax.broadcasted_iota(jnp.int32, (ROWS, COLS), 0) * COLS
                        + lax.broadcasted_iota(jnp.int32, (ROWS, COLS), 1)
                        ).astype(jnp.float32)

        def body(i, d):
            va = d
            ia = lin_ref[...]
            size = ROWS
            while size > 1:
                h = size // 2
                va1, va2 = va[:h], va[h:]
                ia1, ia2 = ia[:h], ia[h:]
                win = (va1 > va2) | ((va1 == va2) & (ia1 < ia2))
                va = jnp.where(win, va1, va2)
                ia = jnp.where(win, ia1, ia2)
                size = h
            mxk = jnp.max(va, axis=1, keepdims=True)
            cand = jnp.where(va == mxk, ia, jnp.float32(3.0e7))
            nxt = jnp.min(cand).astype(jnp.int32)
            idx_out[i] = nxt
            cx = pxs_ref[nxt]
            cy = pys_ref[nxt]
            cz = pzs_ref[nxt]
            row = i >> 7
            lm = lane == (i & 127)
            cen_ref[0, pl.ds(row, 1), :] = jnp.where(
                lm, cx, cen_ref[0, pl.ds(row, 1), :])
            cen_ref[1, pl.ds(row, 1), :] = jnp.where(
                lm, cy, cen_ref[1, pl.ds(row, 1), :])
            cen_ref[2, pl.ds(row, 1), :] = jnp.where(
                lm, cz, cen_ref[2, pl.ds(row, 1), :])
            ddx = px_ref[...] - cx
            ddy = py_ref[...] - cy
            ddz = pz_ref[...] - cz
            dd = ddx * ddx + ddy * ddy + ddz * ddz
            return jnp.minimum(d, dd)

        lax.fori_loop(1, M, body, d0)

    @pl.when(b > 0)
    def _knn():
        bb = b - 1
        cx = jnp.reshape(cen_ref[0, pl.ds(bb, 1), :], (CBLK, 1))
        cy = jnp.reshape(cen_ref[1, pl.ds(bb, 1), :], (CBLK, 1))
        cz = jnp.reshape(cen_ref[2, pl.ds(bb, 1), :], (CBLK, 1))
        px = px1_ref[...]
        py = py1_ref[...]
        pz = pz1_ref[...]
        dx = cx - px
        dy = cy - py
        dz = cz - pz
        d2v = dx * dx + dy * dy + dz * dz
        mv = jnp.min(d2v, axis=1, keepdims=True)

        iotaf = lax.broadcasted_iota(jnp.int32, (CBLK, N), 1).astype(jnp.float32)
        li = lax.broadcasted_iota(jnp.int32, (CBLK, 8), 1)
        acc = jnp.zeros((CBLK, 8), jnp.int32)
        for k in range(K):
            d2 = d2v if k == 0 else d2_ref[...]
            cand = jnp.where(d2 == mv, iotaf, jnp.float32(3.0e7))
            ikf = jnp.min(cand, axis=1, keepdims=True)
            acc = jnp.where(li == k, ikf.astype(jnp.int32), acc)
            if k < K - 1:
                masked = jnp.where(iotaf == ikf, jnp.float32(jnp.inf), d2)
                d2_ref[...] = masked
                mv = jnp.min(masked, axis=1, keepdims=True)
        nbr_out[0] = acc


def _fpsknn_call(pxm, pym, pzm, pxs, pys, pzs, px1, py1, pz1):
    full2 = pl.BlockSpec((ROWS, COLS), lambda b: (0, 0))
    pts_spec = pl.BlockSpec((1, N), lambda b: (0, 0))
    out_shape = [
        jax.ShapeDtypeStruct((MPAD,), jnp.int32),
        jax.ShapeDtypeStruct((NBLK, CBLK, 8), jnp.int32),
    ]
    return pl.pallas_call(
        _fpsknn_body,
        grid=(1 + NBLK,),
        in_specs=[
            full2, full2, full2,
            pl.BlockSpec(memory_space=pltpu.SMEM),
            pl.BlockSpec(memory_space=pltpu.SMEM),
            pl.BlockSpec(memory_space=pltpu.SMEM),
            pts_spec, pts_spec, pts_spec,
        ],
        out_specs=[
            pl.BlockSpec(memory_space=pltpu.SMEM),
            pl.BlockSpec((1, CBLK, 8),
                         lambda b: (jnp.maximum(b - 1, 0), 0, 0)),
        ],
        out_shape=out_shape,
        scratch_shapes=[
            pltpu.VMEM((CBLK, N), jnp.float32),
            pltpu.VMEM((3, 16, COLS), jnp.float32),
            pltpu.VMEM((ROWS, COLS), jnp.float32),
        ],
    )(pxm, pym, pzm, pxs, pys, pzs, px1, py1, pz1)


CCH = MPAD // GCHUNK
TCH = CCH + NCH


def _gather_body(cidx_hbm, gidx_hbm, tx_hbm, ty_hbm, tz_hbm,
                 cx_hbm, cy_hbm, cz_hbm, gx_hbm, gy_hbm, gz_hbm,
                 idx_v, row_v, sem):
    wid = lax.axis_index("s") * 2 + lax.axis_index("c")

    def do_chunk(idx_hbm, outs, c):
        base = c * GCHUNK
        pltpu.sync_copy(idx_hbm.at[pl.ds(base, GCHUNK)], idx_v)
        for t_hbm, o_hbm in zip((tx_hbm, ty_hbm, tz_hbm), outs):
            pltpu.async_copy(t_hbm.at[idx_v], row_v, sem).wait()
            pltpu.sync_copy(row_v, o_hbm.at[pl.ds(base, GCHUNK)])

    for r in range((TCH + NWORK - 1) // NWORK):
        c = wid + r * NWORK

        @pl.when(c < CCH)
        def _():
            do_chunk(cidx_hbm, (cx_hbm, cy_hbm, cz_hbm), c)

        @pl.when((c >= CCH) & (c < TCH))
        def _():
            do_chunk(gidx_hbm, (gx_hbm, gy_hbm, gz_hbm), c - CCH)


def _gather_call(cidx, gidx, px, py, pz):
    mesh = plsc.VectorSubcoreMesh(core_axis_name="c", subcore_axis_name="s")
    f = pl.kernel(
        _gather_body,
        out_type=[jax.ShapeDtypeStruct((MPAD,), jnp.float32)] * 3
        + [jax.ShapeDtypeStruct((GPAD,), jnp.float32)] * 3,
        mesh=mesh,
        scratch_types=[
            pltpu.VMEM((GCHUNK,), jnp.int32),
            pltpu.VMEM((GCHUNK,), jnp.float32),
            pltpu.SemaphoreType.DMA,
        ],
    )
    return f(cidx, gidx, px, py, pz)


def kernel(x, pos, batch):
    px = pos[:, 0]
    py = pos[:, 1]
    pz = pos[:, 2]
    pxm = px.reshape(ROWS, COLS)
    pym = py.reshape(ROWS, COLS)
    pzm = pz.reshape(ROWS, COLS)

    fps_idx, nbr8 = _fpsknn_call(pxm, pym, pzm, px, py, pz,
                                 px.reshape(1, N), py.reshape(1, N),
                                 pz.reshape(1, N))
    gidx = nbr8[:, :, :K].reshape(GPAD)

    ccx, ccy, ccz, ggx, ggy, ggz = _gather_call(fps_idx, gidx, px, py, pz)
    centroids = jnp.stack([ccx[:M], ccy[:M], ccz[:M]], axis=1)
    groups = jnp.stack([ggx[:GTOT], ggy[:GTOT], ggz[:GTOT]], axis=1)
    return centroids, groups
